# baseline (speedup 1.0000x reference)
import functools

import jax
import jax.numpy as jnp
from jax import lax
from jax.experimental import pallas as pl
from jax.experimental.pallas import tpu as pltpu
from jax.experimental.pallas import tpu_sc as plsc

N, E, D, H = 10000, 320000, 128, 128
NC, NS = 2, 16
NW = NC * NS
E_PER_W = E // NW
CHUNK = 80
NCH = E_PER_W // CHUNK
NG = 5
GCH = NCH // NG
RPT = 624
TAIL = N - NS * RPT

_mesh = plsc.VectorSubcoreMesh(core_axis_name="c", subcore_axis_name="s")


@functools.partial(
    pl.kernel,
    out_type=jax.ShapeDtypeStruct((NC, N, D), jnp.float32),
    mesh=_mesh,
    scratch_types=[
        pltpu.VMEM((2, GCH, CHUNK), jnp.int32),
        pltpu.VMEM((2, GCH, CHUNK), jnp.int32),
        pltpu.VMEM((3, CHUNK, D), jnp.float32),
        pltpu.VMEM_SHARED((N, D), jnp.float32),
        pltpu.SemaphoreType.DMA,
        pltpu.SemaphoreType.DMA,
        pltpu.SemaphoreType.DMA,
        pltpu.SemaphoreType.DMA,
        pltpu.SemaphoreType.DMA,
        pltpu.SemaphoreType.DMA,
        pltpu.SemaphoreType.DMA,
    ],
)
def _agg(h_hbm, ei_hbm, out_hbm, src_v, dst_v, rows_v, acc_sh,
         g0, g1, g2, s0, s1, s2, isem):
    cid = lax.axis_index("c")
    sid = lax.axis_index("s")
    wid = sid * NC + cid

    gsem = (g0, g1, g2)
    ssem = (s0, s1, s2)

    def _stage_idx(g, slot):
        pltpu.async_copy(ei_hbm.at[0, wid, g], src_v.at[slot], isem)
        pltpu.async_copy(ei_hbm.at[1, wid, g], dst_v.at[slot], isem)

    def _wait_idx(g, slot):
        pltpu.make_async_copy(ei_hbm.at[0, wid, g], src_v.at[slot], isem).wait()
        pltpu.make_async_copy(ei_hbm.at[1, wid, g], dst_v.at[slot], isem).wait()

    def _gath(slot, j, s):
        pltpu.async_copy(h_hbm.at[src_v.at[slot, j]], rows_v.at[s], gsem[s])

    def _wait_g(slot, j, s):
        pltpu.make_async_copy(h_hbm.at[src_v.at[slot, j]], rows_v.at[s], gsem[s]).wait()

    def _scat(slot, j, s):
        pltpu.async_copy(rows_v.at[s], acc_sh.at[dst_v.at[slot, j]], ssem[s], add=True)

    def _wait_s(slot, j, s):
        pltpu.make_async_copy(rows_v.at[s], acc_sh.at[dst_v.at[slot, j]], ssem[s]).wait()

    _stage_idx(0, 0)

    pltpu.sync_copy(h_hbm.at[pl.ds(sid * RPT, RPT)], acc_sh.at[pl.ds(sid * RPT, RPT)])

    @pl.when(sid == 0)
    def _():
        pltpu.sync_copy(h_hbm.at[pl.ds(NS * RPT, TAIL)], acc_sh.at[pl.ds(NS * RPT, TAIL)])

    _wait_idx(0, 0)
    _gath(0, 0, 0)
    _gath(0, 1, 1)
    plsc.subcore_barrier()

    def group(g, _):
        slot = lax.rem(g, 2)

        @pl.when(g + 1 < NG)
        def _():
            _stage_idx(g + 1, 1 - slot)

        def triple(q, _):
            j0 = 3 * q
            @pl.when(q > 0)
            def _():
                _wait_s(slot, j0 - 1, 2)
            _gath(slot, j0 + 2, 2)
            _wait_g(slot, j0, 0)
            _scat(slot, j0, 0)
            _wait_s(slot, j0, 0)
            _gath(slot, j0 + 3, 0)
            _wait_g(slot, j0 + 1, 1)
            _scat(slot, j0 + 1, 1)
            _wait_s(slot, j0 + 1, 1)

            @pl.when(j0 + 4 < GCH)
            def _():
                _gath(slot, j0 + 4, 1)
            _wait_g(slot, j0 + 2, 2)
            _scat(slot, j0 + 2, 2)
            return 0

        lax.fori_loop(0, GCH // 3, triple, 0)
        _wait_s(slot, GCH - 2, 2)
        _wait_g(slot, GCH - 1, 0)
        _scat(slot, GCH - 1, 0)
        _wait_s(slot, GCH - 1, 0)

        @pl.when(g + 1 < NG)
        def _():
            _wait_idx(g + 1, 1 - slot)
            _gath(1 - slot, 0, 0)
            _gath(1 - slot, 1, 1)
        return 0

    lax.fori_loop(0, NG, group, 0)

    plsc.subcore_barrier()
    pltpu.sync_copy(acc_sh.at[pl.ds(sid * RPT, RPT)],
                    out_hbm.at[cid, pl.ds(sid * RPT, RPT)])

    @pl.when(sid == 0)
    def _():
        pltpu.sync_copy(acc_sh.at[pl.ds(NS * RPT, TAIL)],
                        out_hbm.at[cid, pl.ds(NS * RPT, TAIL)])


BLK = 1000


def _mlp_body(h_ref, p0_ref, p1_ref, w1_ref, b1_ref, w2_ref, b2_ref, o_ref):
    m = p0_ref[0] + p1_ref[0] - h_ref[...]
    t = jnp.dot(m, w1_ref[...], preferred_element_type=jnp.float32) + b1_ref[...]
    t = jnp.maximum(t, 0.0)
    o = jnp.dot(t, w2_ref[...], preferred_element_type=jnp.float32) + b2_ref[...]
    o_ref[...] = jnp.maximum(o, 0.0)


def _mlp(h, p, W1, b1, W2, b2):
    row_spec = pl.BlockSpec((BLK, D), lambda i: (i, 0))
    full = pl.BlockSpec((D, H), lambda i: (0, 0))
    bias = pl.BlockSpec((1, H), lambda i: (0, 0))
    return pl.pallas_call(
        _mlp_body,
        grid=(N // BLK,),
        in_specs=[row_spec,
                  pl.BlockSpec((1, BLK, D), lambda i: (0, i, 0)),
                  pl.BlockSpec((1, BLK, D), lambda i: (1, i, 0)),
                  full, bias, full, bias],
        out_specs=pl.BlockSpec((BLK, H), lambda i: (i, 0)),
        out_shape=jax.ShapeDtypeStruct((N, H), jnp.float32),
    )(h, p, p, W1, b1.reshape(1, H), W2, b2.reshape(1, H))


def kernel(x, edge_index, batch, W1a, b1a, W2a, b2a, W1b, b1b, W2b, b2b):
    ei5 = edge_index.reshape(2, NW, NG, GCH, CHUNK)
    p = _agg(x, ei5)
    h1 = _mlp(x, p, W1a, b1a, W2a, b2a)
    p2 = _agg(h1, ei5)
    return _mlp(h1, p2, W1b, b1b, W2b, b2b)


# speedup vs baseline: 1.0382x; 1.0382x over previous
import functools

import jax
import jax.numpy as jnp
from jax import lax
from jax.experimental import pallas as pl
from jax.experimental.pallas import tpu as pltpu
from jax.experimental.pallas import tpu_sc as plsc

N, E, D, H = 10000, 320000, 128, 128
NC, NS = 2, 16
NW = NC * NS
E_PER_W = E // NW
CHUNK = 80
NCH = E_PER_W // CHUNK
NG = 5
GCH = NCH // NG
RPT = 624
TAIL = N - NS * RPT

_mesh = plsc.VectorSubcoreMesh(core_axis_name="c", subcore_axis_name="s")


@functools.partial(
    pl.kernel,
    out_type=jax.ShapeDtypeStruct((NC, N, D), jnp.float32),
    mesh=_mesh,
    scratch_types=[
        pltpu.VMEM((2, GCH, CHUNK), jnp.int32),
        pltpu.VMEM((2, GCH, CHUNK), jnp.int32),
        pltpu.VMEM((3, CHUNK, D), jnp.float32),
        pltpu.VMEM_SHARED((N, D), jnp.float32),
        pltpu.SemaphoreType.DMA,
        pltpu.SemaphoreType.DMA,
        pltpu.SemaphoreType.DMA,
        pltpu.SemaphoreType.DMA,
        pltpu.SemaphoreType.DMA,
        pltpu.SemaphoreType.DMA,
        pltpu.SemaphoreType.DMA,
    ],
)
def _agg(h_hbm, ei_hbm, out_hbm, src_v, dst_v, rows_v, acc_sh,
         g0, g1, g2, s0, s1, s2, isem):
    cid = lax.axis_index("c")
    sid = lax.axis_index("s")
    wid = sid * NC + cid

    gsem = (g0, g1, g2)
    ssem = (s0, s1, s2)

    def _stage_idx(g, slot):
        pltpu.async_copy(ei_hbm.at[0, wid, g], src_v.at[slot], isem)
        pltpu.async_copy(ei_hbm.at[1, wid, g], dst_v.at[slot], isem)

    def _wait_idx(g, slot):
        pltpu.make_async_copy(ei_hbm.at[0, wid, g], src_v.at[slot], isem).wait()
        pltpu.make_async_copy(ei_hbm.at[1, wid, g], dst_v.at[slot], isem).wait()

    def _gath(slot, j, s):
        pltpu.async_copy(h_hbm.at[src_v.at[slot, j]], rows_v.at[s], gsem[s])

    def _wait_g(slot, j, s):
        pltpu.make_async_copy(h_hbm.at[src_v.at[slot, j]], rows_v.at[s], gsem[s]).wait()

    def _scat(slot, j, s):
        pltpu.async_copy(rows_v.at[s], acc_sh.at[dst_v.at[slot, j]], ssem[s], add=True)

    def _wait_s(slot, j, s):
        pltpu.make_async_copy(rows_v.at[s], acc_sh.at[dst_v.at[slot, j]], ssem[s]).wait()

    _stage_idx(0, 0)

    pltpu.sync_copy(h_hbm.at[pl.ds(sid * RPT, RPT)], acc_sh.at[pl.ds(sid * RPT, RPT)])

    @pl.when(sid == 0)
    def _():
        pltpu.sync_copy(h_hbm.at[pl.ds(NS * RPT, TAIL)], acc_sh.at[pl.ds(NS * RPT, TAIL)])

    _wait_idx(0, 0)
    _gath(0, 0, 0)
    _gath(0, 1, 1)
    plsc.subcore_barrier()

    def group(g, _):
        slot = lax.rem(g, 2)

        @pl.when(g + 1 < NG)
        def _():
            _stage_idx(g + 1, 1 - slot)

        def triple(q, _):
            j0 = 3 * q
            @pl.when(q > 0)
            def _():
                _wait_s(slot, j0 - 1, 2)
            _gath(slot, j0 + 2, 2)
            _wait_g(slot, j0, 0)
            _scat(slot, j0, 0)
            _wait_s(slot, j0, 0)
            _gath(slot, j0 + 3, 0)
            _wait_g(slot, j0 + 1, 1)
            _scat(slot, j0 + 1, 1)
            _wait_s(slot, j0 + 1, 1)

            @pl.when(j0 + 4 < GCH)
            def _():
                _gath(slot, j0 + 4, 1)
            _wait_g(slot, j0 + 2, 2)
            _scat(slot, j0 + 2, 2)
            return 0

        lax.fori_loop(0, GCH // 3, triple, 0)
        _wait_s(slot, GCH - 2, 2)
        _wait_g(slot, GCH - 1, 0)
        _scat(slot, GCH - 1, 0)
        _wait_s(slot, GCH - 1, 0)

        @pl.when(g + 1 < NG)
        def _():
            _wait_idx(g + 1, 1 - slot)
            _gath(1 - slot, 0, 0)
            _gath(1 - slot, 1, 1)
        return 0

    lax.fori_loop(0, NG, group, 0)

    plsc.subcore_barrier()
    pltpu.sync_copy(acc_sh.at[pl.ds(sid * RPT, RPT)],
                    out_hbm.at[cid, pl.ds(sid * RPT, RPT)])

    @pl.when(sid == 0)
    def _():
        pltpu.sync_copy(acc_sh.at[pl.ds(NS * RPT, TAIL)],
                        out_hbm.at[cid, pl.ds(NS * RPT, TAIL)])


BLK = 5000


def _mlp_body(h_ref, p0_ref, p1_ref, w1_ref, b1_ref, w2_ref, b2_ref, o_ref):
    m = p0_ref[0] + p1_ref[0] - h_ref[...]
    t = jnp.dot(m, w1_ref[...], preferred_element_type=jnp.float32) + b1_ref[...]
    t = jnp.maximum(t, 0.0)
    o = jnp.dot(t, w2_ref[...], preferred_element_type=jnp.float32) + b2_ref[...]
    o_ref[...] = jnp.maximum(o, 0.0)


def _mlp(h, p, W1, b1, W2, b2):
    row_spec = pl.BlockSpec((BLK, D), lambda i: (i, 0))
    full = pl.BlockSpec((D, H), lambda i: (0, 0))
    bias = pl.BlockSpec((1, H), lambda i: (0, 0))
    return pl.pallas_call(
        _mlp_body,
        grid=(N // BLK,),
        in_specs=[row_spec,
                  pl.BlockSpec((1, BLK, D), lambda i: (0, i, 0)),
                  pl.BlockSpec((1, BLK, D), lambda i: (1, i, 0)),
                  full, bias, full, bias],
        out_specs=pl.BlockSpec((BLK, H), lambda i: (i, 0)),
        out_shape=jax.ShapeDtypeStruct((N, H), jnp.float32),
    )(h, p, p, W1, b1.reshape(1, H), W2, b2.reshape(1, H))


def kernel(x, edge_index, batch, W1a, b1a, W2a, b2a, W1b, b1b, W2b, b2b):
    ei5 = edge_index.reshape(2, NW, NG, GCH, CHUNK)
    p = _agg(x, ei5)
    h1 = _mlp(x, p, W1a, b1a, W2a, b2a)
    p2 = _agg(h1, ei5)
    return _mlp(h1, p2, W1b, b1b, W2b, b2b)
